# baseline (speedup 1.0000x reference)
import jax
import jax.numpy as jnp
from jax import lax
from jax.experimental import pallas as pl
from jax.experimental.pallas import tpu as pltpu
from jax.experimental.pallas import tpu_sc as plsc

R = 128
N = 32768
K = 32
L = 16
NC, NS = 2, 16
NW = NC * NS
RPW = R // NW
NCHUNK = N // L
BLK = 8
NBLK = NCHUNK // BLK
CAP = 2048
HCAP = 32
HTRASH = L * HCAP
CAPL = 64
CTRASH = L * CAPL

_NEG_INF = float("-inf")
_BIG_IDX = 2**30


def _row_topk(row_v, bmax_v, hitg_v, cval2_v, cidx2_v, cval_v, cidx_v, oval_v, oidx_v):
    def p1_body(bi, accs):
        base = bi * (BLK * L)
        c = [row_v[pl.ds(base + j * L, L)] for j in range(BLK)]
        m01 = jnp.maximum(jnp.maximum(c[0], c[1]),
                          jnp.maximum(c[2], c[3]))
        m23 = jnp.maximum(jnp.maximum(c[4], c[5]),
                          jnp.maximum(c[6], c[7]))
        bmax_v[pl.ds(bi * L, L)] = jnp.maximum(m01, m23)
        return tuple(jnp.maximum(accs[j], c[j]) for j in range(BLK))

    ninf = jnp.full((L,), _NEG_INF, dtype=jnp.float32)
    accs = lax.fori_loop(0, NBLK, p1_body, (ninf,) * BLK, unroll=2)

    def sort16(v):
        return plsc.sort_key_val(v, v, descending=True)[0]

    def merge_top32(t1, t2, s):
        rs = lax.rev(s, (0,))
        hi = sort16(jnp.maximum(t1, rs))
        lo = sort16(jnp.minimum(t1, rs))
        nxt = sort16(jnp.maximum(t2, lax.rev(lo, (0,))))
        return hi, nxt

    s0 = sort16(accs[0])
    rs1 = lax.rev(sort16(accs[1]), (0,))
    t1 = sort16(jnp.maximum(s0, rs1))
    t2 = sort16(jnp.minimum(s0, rs1))
    for j in range(2, BLK):
        t1, t2 = merge_top32(t1, t2, sort16(accs[j]))
    t = jnp.take(t2, jnp.full((L,), L - 1, dtype=jnp.int32))

    lane_iota = lax.iota(jnp.int32, L)
    zi = jnp.zeros((L,), dtype=jnp.int32)

    def hits_body(g, hcnt):
        mask = bmax_v[pl.ds(g * L, L)] >= t
        dest = jnp.where(mask, lane_iota * HCAP + hcnt, HTRASH + lane_iota)
        plsc.store_scatter(hitg_v, [dest], zi + g)
        return jnp.minimum(hcnt + mask.astype(jnp.int32), HCAP)

    hcnt = lax.fori_loop(0, NBLK, hits_body, zi, unroll=2)

    def xlane_max(v):
        for sh in (8, 4, 2, 1):
            v = jnp.maximum(v, jnp.take(v, lane_iota ^ sh))
        return v

    hmax = xlane_max(hcnt)[0]

    def q_body(q, ccnt):
        valid_q = q < hcnt
        g = plsc.load_gather(hitg_v, [lane_iota * HCAP + q])
        gbase = jnp.minimum(jnp.maximum(g, 0), NBLK - 1) * (BLK * L)
        out = ccnt
        for i in range(BLK):
            idx = gbase + i * L + lane_iota
            v = plsc.load_gather(row_v, [idx])
            mask = (v >= t) & valid_q
            dest = jnp.where(mask, lane_iota * CAPL + out, CTRASH + lane_iota)
            plsc.store_scatter(cval2_v, [dest], v)
            plsc.store_scatter(cidx2_v, [dest], idx)
            out = jnp.minimum(out + mask.astype(jnp.int32), CAPL)
        return out

    ccnt = lax.fori_loop(0, hmax, q_body, zi)

    ps = ccnt
    for sh in (1, 2, 4, 8):
        prev = jnp.take(ps, jnp.maximum(lane_iota - sh, 0))
        ps = ps + jnp.where(lane_iota >= sh, prev, 0)
    excl = ps - ccnt
    cmax = xlane_max(ccnt)[0]
    n = jnp.take(ps, jnp.full((L,), L - 1, dtype=jnp.int32))[0]

    def reloc_body(q, _):
        src = lane_iota * CAPL + q
        v = plsc.load_gather(cval2_v, [src])
        ii = plsc.load_gather(cidx2_v, [src])
        valid = q < ccnt
        dest = jnp.where(valid, excl + q, CAP + lane_iota)
        plsc.store_scatter(cval_v, [dest], v)
        plsc.store_scatter(cidx_v, [dest], ii)
        return 0

    lax.fori_loop(0, cmax, reloc_body, 0)
    cval_v[pl.ds(n, L)] = ninf
    nv = (n + L - 1) // L

    def sel_step(k, carry):
        pv, pi, ov0, ov1, oi0, oi1 = carry

        def scan_vregs(j, best):
            bv, bi = best
            v = cval_v[pl.ds(j * L, L)]
            ii = cidx_v[pl.ds(j * L, L)]
            elig = (v < pv) | ((v == pv) & (ii > pi))
            v2 = jnp.where(elig, v, _NEG_INF)
            take = (v2 > bv) | ((v2 == bv) & (ii < bi))
            return (jnp.where(take, v2, bv), jnp.where(take, ii, bi))

        binit = (ninf, jnp.full((L,), _BIG_IDX, dtype=jnp.int32))
        bv, bi = lax.fori_loop(0, nv, scan_vregs, binit)
        for sh in (8, 4, 2, 1):
            perm = lane_iota ^ sh
            vg = jnp.take(bv, perm)
            ig = jnp.take(bi, perm)
            better = (vg > bv) | ((vg == bv) & (ig < bi))
            bv = jnp.where(better, vg, bv)
            bi = jnp.where(better, ig, bi)
        slot0 = (k < L) & (lane_iota == k)
        slot1 = (k >= L) & (lane_iota == k - L)
        ov0 = jnp.where(slot0, bv, ov0)
        ov1 = jnp.where(slot1, bv, ov1)
        oi0 = jnp.where(slot0, bi, oi0)
        oi1 = jnp.where(slot1, bi, oi1)
        return (bv, bi, ov0, ov1, oi0, oi1)

    zf = jnp.zeros((L,), dtype=jnp.float32)
    zi = jnp.zeros((L,), dtype=jnp.int32)
    pinf = jnp.full((L,), float("inf"), dtype=jnp.float32)
    _, _, ov0, ov1, oi0, oi1 = lax.fori_loop(
        0, K, sel_step,
        (pinf, jnp.full((L,), -1, dtype=jnp.int32), zf, zf, zi, zi))
    oval_v[pl.ds(0, L)] = ov0
    oval_v[pl.ds(L, L)] = ov1
    oidx_v[pl.ds(0, L)] = oi0
    oidx_v[pl.ds(L, L)] = oi1


def _topk_body(x_hbm, val_hbm, idx_hbm,
               row0_v, row1_v, bmax_v, hitg_v, cval2_v, cidx2_v,
               cval_v, cidx_v, oval_v, oidx_v, sem0, sem1):
    wid = lax.axis_index("s") * NC + lax.axis_index("c")
    base_row = wid * RPW
    bufs = (row0_v, row1_v)
    sems = (sem0, sem1)

    handles = {0: pltpu.async_copy(x_hbm.at[base_row], row0_v, sem0)}
    for r in range(RPW):
        if r + 1 < RPW:
            handles[(r + 1) % 2] = pltpu.async_copy(
                x_hbm.at[base_row + r + 1], bufs[(r + 1) % 2],
                sems[(r + 1) % 2])
        handles[r % 2].wait()
        _row_topk(bufs[r % 2], bmax_v, hitg_v, cval2_v, cidx2_v, cval_v, cidx_v, oval_v, oidx_v)
        pltpu.sync_copy(oval_v, val_hbm.at[base_row + r])
        pltpu.sync_copy(oidx_v, idx_hbm.at[base_row + r])


@jax.jit
def kernel(x):
    mesh = plsc.VectorSubcoreMesh(
        core_axis_name="c", subcore_axis_name="s",
        num_cores=NC, num_subcores=NS)
    f = pl.kernel(
        _topk_body,
        out_type=(
            jax.ShapeDtypeStruct((R, K), jnp.float32),
            jax.ShapeDtypeStruct((R, K), jnp.int32),
        ),
        compiler_params=pltpu.CompilerParams(needs_layout_passes=False),
        mesh=mesh,
        scratch_types=[
            pltpu.VMEM((N,), jnp.float32),
            pltpu.VMEM((N,), jnp.float32),
            pltpu.VMEM((NBLK * L,), jnp.float32),
            pltpu.VMEM((L * HCAP + L,), jnp.int32),
            pltpu.VMEM((L * CAPL + L,), jnp.float32),
            pltpu.VMEM((L * CAPL + L,), jnp.int32),
            pltpu.VMEM((CAP + L,), jnp.float32),
            pltpu.VMEM((CAP + L,), jnp.int32),
            pltpu.VMEM((K,), jnp.float32),
            pltpu.VMEM((K,), jnp.int32),
            pltpu.SemaphoreType.DMA,
            pltpu.SemaphoreType.DMA,
        ],
    )
    return f(x)


# speedup vs baseline: 1.0170x; 1.0170x over previous
import jax
import jax.numpy as jnp
from jax import lax
from jax.experimental import pallas as pl
from jax.experimental.pallas import tpu as pltpu
from jax.experimental.pallas import tpu_sc as plsc

R = 128
N = 32768
K = 32
L = 16
NC, NS = 2, 16
NW = NC * NS
RPW = R // NW
NCHUNK = N // L
BLK = 8
NBLK = NCHUNK // BLK
CAP = 2048
HCAP = 32
HTRASH = L * HCAP
CAPL = 64
CTRASH = L * CAPL

_NEG_INF = float("-inf")
_BIG_IDX = 2**30


def _row_topk(row_v, bmax_v, hitg_v, cval2_v, cidx2_v, cval_v, cidx_v, oval_v, oidx_v):
    def p1_body(bi, accs):
        base = bi * (BLK * L)
        c = [row_v[pl.ds(base + j * L, L)] for j in range(BLK)]
        p0 = jnp.maximum(c[0], c[1])
        p1 = jnp.maximum(c[2], c[3])
        p2 = jnp.maximum(c[4], c[5])
        p3 = jnp.maximum(c[6], c[7])
        bmax_v[pl.ds(bi * L, L)] = jnp.maximum(jnp.maximum(p0, p1),
                                               jnp.maximum(p2, p3))
        return (jnp.maximum(accs[0], p0), jnp.maximum(accs[1], p1),
                jnp.maximum(accs[2], p2), jnp.maximum(accs[3], p3))

    ninf = jnp.full((L,), _NEG_INF, dtype=jnp.float32)
    accs = lax.fori_loop(0, NBLK, p1_body, (ninf,) * 4, unroll=2)

    def sort16(v):
        return plsc.sort_key_val(v, v, descending=True)[0]

    def merge_top32(t1, t2, s):
        rs = lax.rev(s, (0,))
        hi = sort16(jnp.maximum(t1, rs))
        lo = sort16(jnp.minimum(t1, rs))
        nxt = sort16(jnp.maximum(t2, lax.rev(lo, (0,))))
        return hi, nxt

    s0 = sort16(accs[0])
    rs1 = lax.rev(sort16(accs[1]), (0,))
    t1 = sort16(jnp.maximum(s0, rs1))
    t2 = sort16(jnp.minimum(s0, rs1))
    for j in range(2, 4):
        t1, t2 = merge_top32(t1, t2, sort16(accs[j]))
    t = jnp.take(t2, jnp.full((L,), L - 1, dtype=jnp.int32))

    lane_iota = lax.iota(jnp.int32, L)
    zi = jnp.zeros((L,), dtype=jnp.int32)

    def hits_body(g, hcnt):
        mask = bmax_v[pl.ds(g * L, L)] >= t
        dest = jnp.where(mask, lane_iota * HCAP + hcnt, HTRASH + lane_iota)
        plsc.store_scatter(hitg_v, [dest], zi + g)
        return jnp.minimum(hcnt + mask.astype(jnp.int32), HCAP)

    hcnt = lax.fori_loop(0, NBLK, hits_body, zi, unroll=2)

    def xlane_max(v):
        for sh in (8, 4, 2, 1):
            v = jnp.maximum(v, jnp.take(v, lane_iota ^ sh))
        return v

    hmax = xlane_max(hcnt)[0]

    def q_body(q, ccnt):
        valid_q = q < hcnt
        g = plsc.load_gather(hitg_v, [lane_iota * HCAP + q])
        gbase = jnp.minimum(jnp.maximum(g, 0), NBLK - 1) * (BLK * L)
        out = ccnt
        for i in range(BLK):
            idx = gbase + i * L + lane_iota
            v = plsc.load_gather(row_v, [idx])
            mask = (v >= t) & valid_q
            dest = jnp.where(mask, lane_iota * CAPL + out, CTRASH + lane_iota)
            plsc.store_scatter(cval2_v, [dest], v)
            plsc.store_scatter(cidx2_v, [dest], idx)
            out = jnp.minimum(out + mask.astype(jnp.int32), CAPL)
        return out

    ccnt = lax.fori_loop(0, hmax, q_body, zi)

    ps = ccnt
    for sh in (1, 2, 4, 8):
        prev = jnp.take(ps, jnp.maximum(lane_iota - sh, 0))
        ps = ps + jnp.where(lane_iota >= sh, prev, 0)
    excl = ps - ccnt
    cmax = xlane_max(ccnt)[0]
    n = jnp.take(ps, jnp.full((L,), L - 1, dtype=jnp.int32))[0]

    def reloc_body(q, _):
        src = lane_iota * CAPL + q
        v = plsc.load_gather(cval2_v, [src])
        ii = plsc.load_gather(cidx2_v, [src])
        valid = q < ccnt
        dest = jnp.where(valid, excl + q, CAP + lane_iota)
        plsc.store_scatter(cval_v, [dest], v)
        plsc.store_scatter(cidx_v, [dest], ii)
        return 0

    lax.fori_loop(0, cmax, reloc_body, 0)
    cval_v[pl.ds(n, L)] = ninf
    nv = (n + L - 1) // L

    def sel_step(k, carry):
        pv, pi, ov0, ov1, oi0, oi1 = carry

        def scan_vregs(j, best):
            bv, bi = best
            v = cval_v[pl.ds(j * L, L)]
            ii = cidx_v[pl.ds(j * L, L)]
            elig = (v < pv) | ((v == pv) & (ii > pi))
            v2 = jnp.where(elig, v, _NEG_INF)
            take = (v2 > bv) | ((v2 == bv) & (ii < bi))
            return (jnp.where(take, v2, bv), jnp.where(take, ii, bi))

        binit = (ninf, jnp.full((L,), _BIG_IDX, dtype=jnp.int32))
        bv, bi = lax.fori_loop(0, nv, scan_vregs, binit)
        for sh in (8, 4, 2, 1):
            perm = lane_iota ^ sh
            vg = jnp.take(bv, perm)
            ig = jnp.take(bi, perm)
            better = (vg > bv) | ((vg == bv) & (ig < bi))
            bv = jnp.where(better, vg, bv)
            bi = jnp.where(better, ig, bi)
        slot0 = (k < L) & (lane_iota == k)
        slot1 = (k >= L) & (lane_iota == k - L)
        ov0 = jnp.where(slot0, bv, ov0)
        ov1 = jnp.where(slot1, bv, ov1)
        oi0 = jnp.where(slot0, bi, oi0)
        oi1 = jnp.where(slot1, bi, oi1)
        return (bv, bi, ov0, ov1, oi0, oi1)

    zf = jnp.zeros((L,), dtype=jnp.float32)
    zi = jnp.zeros((L,), dtype=jnp.int32)
    pinf = jnp.full((L,), float("inf"), dtype=jnp.float32)
    _, _, ov0, ov1, oi0, oi1 = lax.fori_loop(
        0, K, sel_step,
        (pinf, jnp.full((L,), -1, dtype=jnp.int32), zf, zf, zi, zi))
    oval_v[pl.ds(0, L)] = ov0
    oval_v[pl.ds(L, L)] = ov1
    oidx_v[pl.ds(0, L)] = oi0
    oidx_v[pl.ds(L, L)] = oi1


def _topk_body(x_hbm, val_hbm, idx_hbm,
               row0_v, row1_v, bmax_v, hitg_v, cval2_v, cidx2_v,
               cval_v, cidx_v, oval_v, oidx_v, sem0, sem1):
    wid = lax.axis_index("s") * NC + lax.axis_index("c")
    base_row = wid * RPW
    bufs = (row0_v, row1_v)
    sems = (sem0, sem1)

    handles = {0: pltpu.async_copy(x_hbm.at[base_row], row0_v, sem0)}
    for r in range(RPW):
        if r + 1 < RPW:
            handles[(r + 1) % 2] = pltpu.async_copy(
                x_hbm.at[base_row + r + 1], bufs[(r + 1) % 2],
                sems[(r + 1) % 2])
        handles[r % 2].wait()
        _row_topk(bufs[r % 2], bmax_v, hitg_v, cval2_v, cidx2_v, cval_v,
                  cidx_v, oval_v.at[r], oidx_v.at[r])
    pltpu.sync_copy(oval_v, val_hbm.at[pl.ds(base_row, RPW)])
    pltpu.sync_copy(oidx_v, idx_hbm.at[pl.ds(base_row, RPW)])


@jax.jit
def kernel(x):
    mesh = plsc.VectorSubcoreMesh(
        core_axis_name="c", subcore_axis_name="s",
        num_cores=NC, num_subcores=NS)
    f = pl.kernel(
        _topk_body,
        out_type=(
            jax.ShapeDtypeStruct((R, K), jnp.float32),
            jax.ShapeDtypeStruct((R, K), jnp.int32),
        ),
        compiler_params=pltpu.CompilerParams(needs_layout_passes=False),
        mesh=mesh,
        scratch_types=[
            pltpu.VMEM((N,), jnp.float32),
            pltpu.VMEM((N,), jnp.float32),
            pltpu.VMEM((NBLK * L,), jnp.float32),
            pltpu.VMEM((L * HCAP + L,), jnp.int32),
            pltpu.VMEM((L * CAPL + L,), jnp.float32),
            pltpu.VMEM((L * CAPL + L,), jnp.int32),
            pltpu.VMEM((CAP + L,), jnp.float32),
            pltpu.VMEM((CAP + L,), jnp.int32),
            pltpu.VMEM((RPW, K), jnp.float32),
            pltpu.VMEM((RPW, K), jnp.int32),
            pltpu.SemaphoreType.DMA,
            pltpu.SemaphoreType.DMA,
        ],
    )
    return f(x)
